# Initial kernel scaffold; baseline (speedup 1.0000x reference)
import jax
import jax.numpy as jnp
from jax.experimental import pallas as pl


def kernel(x, edge_index, batch, W1, b1, W2, b2, Wp1, bp1, Wp2, bp2):
    raise NotImplementedError("write your pallas kernel here")


# speedup vs baseline: 5.2618x; 5.2618x over previous
import functools

import jax
import jax.numpy as jnp
from jax import lax
from jax.experimental import pallas as pl
from jax.experimental.pallas import tpu as pltpu
from jax.experimental.pallas import tpu_sc as plsc

N = 10000
D = 128
G = 128
P = 64
E = 320000

NC = 2
NS = 16
NW = NC * NS
EPW = E // NW
CHUNK = 80
NCHUNK = EPW // CHUNK
NPAD = 10240
RPT = NPAD // NS
ZROWS = 128

@functools.cache
def _sc_segment_sum_fn():
    mesh = plsc.VectorSubcoreMesh(
        core_axis_name="c", subcore_axis_name="s",
        num_cores=NC, num_subcores=NS)
    return functools.partial(
        pl.kernel,
        out_type=jax.ShapeDtypeStruct((NC, NPAD, D), jnp.float32),
        mesh=mesh,
        scratch_types=[
            pltpu.VMEM_SHARED((NPAD, D), jnp.float32),
            pltpu.VMEM((CHUNK,), jnp.int32),
            pltpu.VMEM((CHUNK,), jnp.int32),
            pltpu.VMEM((CHUNK, D), jnp.float32),
            pltpu.VMEM((ZROWS, D), jnp.float32),
            pltpu.SemaphoreType.DMA,
        ],
    )(_sc_segment_sum_body)


def _sc_segment_sum(feat, src, dst):
    return _sc_segment_sum_fn()(feat, src, dst)


def _sc_segment_sum_body(feat_hbm, src_hbm, dst_hbm, out_hbm,
                         acc_sh, src_v, dst_v, rows_v, zbuf, sem):
    c = lax.axis_index("c")
    s = lax.axis_index("s")
    wid = c * NS + s

    def _zb(i, carry):
        zbuf[i // 8, pl.ds((i % 8) * 16, 16)] = jnp.zeros((16,), jnp.float32)
        return carry
    lax.fori_loop(0, ZROWS * 8, _zb, 0)
    for k in range(RPT // ZROWS):
        pltpu.sync_copy(zbuf, acc_sh.at[pl.ds(s * RPT + k * ZROWS, ZROWS)])
    plsc.subcore_barrier()

    base = wid * EPW

    def _edges(i, carry):
        off = base + i * CHUNK
        pltpu.sync_copy(src_hbm.at[pl.ds(off, CHUNK)], src_v)
        pltpu.sync_copy(dst_hbm.at[pl.ds(off, CHUNK)], dst_v)
        pltpu.async_copy(feat_hbm.at[src_v], rows_v, sem).wait()
        pltpu.sync_copy(rows_v, acc_sh.at[dst_v], add=True)
        return carry
    lax.fori_loop(0, NCHUNK, _edges, 0)
    plsc.subcore_barrier()
    pltpu.sync_copy(acc_sh.at[pl.ds(s * RPT, RPT)],
                    out_hbm.at[c, pl.ds(s * RPT, RPT)])


BLK = 1000
NBLK = N // BLK


def _tc_layer1_body(x_ref, a_ref, w_ref, b_ref, o_ref):
    acc = x_ref[...] + a_ref[0] + a_ref[1]
    h = jnp.dot(acc, w_ref[...], preferred_element_type=jnp.float32) + b_ref[...]
    o_ref[...] = jnp.maximum(h, 0.0)


def _tc_layer1(x, agg, w, b):
    return pl.pallas_call(
        _tc_layer1_body,
        grid=(NBLK,),
        in_specs=[
            pl.BlockSpec((BLK, D), lambda i: (i, 0)),
            pl.BlockSpec((NC, BLK, D), lambda i: (0, i, 0)),
            pl.BlockSpec((D, D), lambda i: (0, 0)),
            pl.BlockSpec((1, D), lambda i: (0, 0)),
        ],
        out_specs=pl.BlockSpec((BLK, D), lambda i: (i, 0)),
        out_shape=jax.ShapeDtypeStruct((N, D), jnp.float32),
    )(x, agg, w, b)


def _tc_layer2_body(h_ref, a_ref, w_ref, b_ref, batch_ref, ps_ref, cnt_ref):
    i = pl.program_id(0)
    acc = h_ref[...] + a_ref[0] + a_ref[1]
    h2 = jnp.maximum(
        jnp.dot(acc, w_ref[...], preferred_element_type=jnp.float32) + b_ref[...],
        0.0)
    b2d = batch_ref[0]
    onehot_t = (lax.broadcasted_iota(jnp.int32, (G, 1), 0) == b2d
                ).astype(jnp.float32)
    ps = jnp.dot(onehot_t, h2, preferred_element_type=jnp.float32)
    cnt = jnp.dot(onehot_t, jnp.ones((BLK, D), jnp.float32),
                  preferred_element_type=jnp.float32)

    @pl.when(i == 0)
    def _():
        ps_ref[...] = jnp.zeros_like(ps_ref)
        cnt_ref[...] = jnp.zeros_like(cnt_ref)

    ps_ref[...] += ps
    cnt_ref[...] += cnt


def _tc_layer2(h, agg, w, b, batch3):
    return pl.pallas_call(
        _tc_layer2_body,
        grid=(NBLK,),
        in_specs=[
            pl.BlockSpec((BLK, D), lambda i: (i, 0)),
            pl.BlockSpec((NC, BLK, D), lambda i: (0, i, 0)),
            pl.BlockSpec((D, D), lambda i: (0, 0)),
            pl.BlockSpec((1, D), lambda i: (0, 0)),
            pl.BlockSpec((1, 1, BLK), lambda i: (i, 0, 0)),
        ],
        out_specs=[
            pl.BlockSpec((G, D), lambda i: (0, 0)),
            pl.BlockSpec((G, D), lambda i: (0, 0)),
        ],
        out_shape=[
            jax.ShapeDtypeStruct((G, D), jnp.float32),
            jax.ShapeDtypeStruct((G, D), jnp.float32),
        ],
    )(h, agg, w, b, batch3)


def _tc_head_body(ps_ref, cnt_ref, wp1_ref, bp1_ref, wp2_ref, bp2_ref, z_ref):
    hg = ps_ref[...] / jnp.maximum(cnt_ref[...], 1.0)
    t = jnp.maximum(
        jnp.dot(hg, wp1_ref[...], preferred_element_type=jnp.float32)
        + bp1_ref[...], 0.0)
    z_ref[...] = (jnp.dot(t, wp2_ref[...], preferred_element_type=jnp.float32)
                  + bp2_ref[...])


def _tc_head(ps, cnt, wp1, bp1, wp2, bp2):
    return pl.pallas_call(
        _tc_head_body,
        out_shape=jax.ShapeDtypeStruct((G, P), jnp.float32),
    )(ps, cnt, wp1, bp1, wp2, bp2)


def kernel(x, edge_index, batch, W1, b1, W2, b2, Wp1, bp1, Wp2, bp2):
    src = edge_index[0]
    dst = edge_index[1]
    batch3 = batch.reshape(NBLK, 1, BLK)
    agg1 = _sc_segment_sum(x, src, dst)
    h = _tc_layer1(x, agg1, W1, b1.reshape(1, D))
    agg2 = _sc_segment_sum(h, src, dst)
    ps, cnt = _tc_layer2(h, agg2, W2, b2.reshape(1, D), batch3)
    return _tc_head(ps, cnt, Wp1, bp1.reshape(1, D), Wp2, bp2.reshape(1, P))


# speedup vs baseline: 10.2004x; 1.9386x over previous
import functools

import jax
import jax.numpy as jnp
from jax import lax
from jax.experimental import pallas as pl
from jax.experimental.pallas import tpu as pltpu
from jax.experimental.pallas import tpu_sc as plsc

N = 10000
D = 128
G = 128
P = 64
E = 320000

NC = 2
NS = 16
NW = NC * NS
EPW = E // NW
CHUNK = 40
NCHUNK = EPW // CHUNK
NPAD = 10240
RPT = NPAD // NS
RING = 5

@functools.cache
def _sc_segment_sum_fn():
    mesh = plsc.VectorSubcoreMesh(
        core_axis_name="c", subcore_axis_name="s",
        num_cores=NC, num_subcores=NS)
    return functools.partial(
        pl.kernel,
        out_type=jax.ShapeDtypeStruct((NC, NPAD, D), jnp.float32),
        mesh=mesh,
        scratch_types=[
            pltpu.VMEM_SHARED((NPAD, D), jnp.float32),
            [pltpu.VMEM((CHUNK,), jnp.int32) for _ in range(RING)],
            [pltpu.VMEM((CHUNK,), jnp.int32) for _ in range(RING)],
            pltpu.VMEM((RING, CHUNK, D), jnp.float32),
            pltpu.SemaphoreType.DMA,
            pltpu.SemaphoreType.DMA,
            pltpu.SemaphoreType.DMA,
        ],
    )(_sc_segment_sum_body)


def _sc_segment_sum(feat, src, dst):
    return _sc_segment_sum_fn()(feat, src, dst)


def _sc_segment_sum_body(feat_hbm, src_hbm, dst_hbm, out_hbm,
                         acc_sh, sbufs, dbufs, rows_v, isem, gsem, ssem):
    c = lax.axis_index("c")
    s = lax.axis_index("s")
    wid = c * NS + s
    base = wid * EPW

    def _zb(i, carry):
        rows_v[0, i // 8, pl.ds((i % 8) * 16, 16)] = jnp.zeros((16,), jnp.float32)
        return carry
    lax.fori_loop(0, CHUNK * 8, _zb, 0)
    for t in range(RPT // CHUNK):
        pltpu.async_copy(rows_v.at[0],
                         acc_sh.at[pl.ds(s * RPT + t * CHUNK, CHUNK)], ssem)
    for t in range(RPT // CHUNK):
        pltpu.make_async_copy(rows_v.at[0],
                              acc_sh.at[pl.ds(s * RPT + t * CHUNK, CHUNK)],
                              ssem).wait()
    plsc.subcore_barrier()

    def _i_issue(j, r):
        off = base + j * CHUNK
        pltpu.async_copy(src_hbm.at[pl.ds(off, CHUNK)], sbufs[r], isem)
        pltpu.async_copy(dst_hbm.at[pl.ds(off, CHUNK)], dbufs[r], isem)

    def _i_drain(j, r):
        off = base + j * CHUNK
        pltpu.make_async_copy(src_hbm.at[pl.ds(off, CHUNK)], sbufs[r],
                              isem).wait()
        pltpu.make_async_copy(dst_hbm.at[pl.ds(off, CHUNK)], dbufs[r],
                              isem).wait()

    def _g_issue(r):
        pltpu.async_copy(feat_hbm.at[sbufs[r]], rows_v.at[r], gsem)

    def _g_drain(r):
        pltpu.make_async_copy(feat_hbm.at[sbufs[r]], rows_v.at[r],
                              gsem).wait()

    def _s_issue(r):
        pltpu.async_copy(rows_v.at[r], acc_sh.at[dbufs[r]], ssem, add=True)

    def _s_drain(r):
        pltpu.make_async_copy(rows_v.at[r], acc_sh.at[dbufs[r]],
                              ssem).wait()

    for j in range(3):
        _i_issue(j, j)
    for j in range(2):
        _i_drain(j, j)
        _g_issue(j)
    for j in range(2):
        _g_drain(j)
        _s_issue(j)
        _i_issue(j + 3, j + 3)
        _i_drain(j + 2, j + 2)
        _g_issue(j + 2)

    def _steady(k, carry):
        j0 = 2 + RING * k
        for u in range(RING):
            j = j0 + u
            r = (2 + u) % RING
            _g_drain(r)
            _s_issue(r)
            _s_drain(u % RING)
            _i_issue(j + 3, u % RING)
            _i_drain(j + 2, (4 + u) % RING)
            _g_issue((4 + u) % RING)
        return carry
    lax.fori_loop(0, (NCHUNK - RING) // RING, _steady, 0)

    for j in range(NCHUNK - 3, NCHUNK):
        _g_drain(j % RING)
        _s_issue(j % RING)
        _s_drain((j - 2) % RING)
        if j + 2 < NCHUNK:
            _i_drain(j + 2, (j + 2) % RING)
            _g_issue((j + 2) % RING)
    for j in range(NCHUNK - 2, NCHUNK):
        _s_drain(j % RING)

    plsc.subcore_barrier()
    pltpu.sync_copy(acc_sh.at[pl.ds(s * RPT, RPT)],
                    out_hbm.at[c, pl.ds(s * RPT, RPT)])


BLK = 1000
NBLK = N // BLK


def _tc_layer1_body(x_ref, a_ref, w_ref, b_ref, o_ref):
    acc = x_ref[...] + a_ref[0] + a_ref[1]
    h = jnp.dot(acc, w_ref[...], preferred_element_type=jnp.float32) + b_ref[...]
    o_ref[...] = jnp.maximum(h, 0.0)


def _tc_layer1(x, agg, w, b):
    return pl.pallas_call(
        _tc_layer1_body,
        grid=(NBLK,),
        in_specs=[
            pl.BlockSpec((BLK, D), lambda i: (i, 0)),
            pl.BlockSpec((NC, BLK, D), lambda i: (0, i, 0)),
            pl.BlockSpec((D, D), lambda i: (0, 0)),
            pl.BlockSpec((1, D), lambda i: (0, 0)),
        ],
        out_specs=pl.BlockSpec((BLK, D), lambda i: (i, 0)),
        out_shape=jax.ShapeDtypeStruct((N, D), jnp.float32),
    )(x, agg, w, b)


def _tc_layer2_body(h_ref, a_ref, w_ref, b_ref, batch_ref, ps_ref, cnt_ref):
    i = pl.program_id(0)
    acc = h_ref[...] + a_ref[0] + a_ref[1]
    h2 = jnp.maximum(
        jnp.dot(acc, w_ref[...], preferred_element_type=jnp.float32) + b_ref[...],
        0.0)
    b2d = batch_ref[0]
    onehot_t = (lax.broadcasted_iota(jnp.int32, (G, 1), 0) == b2d
                ).astype(jnp.float32)
    ps = jnp.dot(onehot_t, h2, preferred_element_type=jnp.float32)
    cnt = jnp.dot(onehot_t, jnp.ones((BLK, D), jnp.float32),
                  preferred_element_type=jnp.float32)

    @pl.when(i == 0)
    def _():
        ps_ref[...] = jnp.zeros_like(ps_ref)
        cnt_ref[...] = jnp.zeros_like(cnt_ref)

    ps_ref[...] += ps
    cnt_ref[...] += cnt


def _tc_layer2(h, agg, w, b, batch3):
    return pl.pallas_call(
        _tc_layer2_body,
        grid=(NBLK,),
        in_specs=[
            pl.BlockSpec((BLK, D), lambda i: (i, 0)),
            pl.BlockSpec((NC, BLK, D), lambda i: (0, i, 0)),
            pl.BlockSpec((D, D), lambda i: (0, 0)),
            pl.BlockSpec((1, D), lambda i: (0, 0)),
            pl.BlockSpec((1, 1, BLK), lambda i: (i, 0, 0)),
        ],
        out_specs=[
            pl.BlockSpec((G, D), lambda i: (0, 0)),
            pl.BlockSpec((G, D), lambda i: (0, 0)),
        ],
        out_shape=[
            jax.ShapeDtypeStruct((G, D), jnp.float32),
            jax.ShapeDtypeStruct((G, D), jnp.float32),
        ],
    )(h, agg, w, b, batch3)


def _tc_head_body(ps_ref, cnt_ref, wp1_ref, bp1_ref, wp2_ref, bp2_ref, z_ref):
    hg = ps_ref[...] / jnp.maximum(cnt_ref[...], 1.0)
    t = jnp.maximum(
        jnp.dot(hg, wp1_ref[...], preferred_element_type=jnp.float32)
        + bp1_ref[...], 0.0)
    z_ref[...] = (jnp.dot(t, wp2_ref[...], preferred_element_type=jnp.float32)
                  + bp2_ref[...])


def _tc_head(ps, cnt, wp1, bp1, wp2, bp2):
    return pl.pallas_call(
        _tc_head_body,
        out_shape=jax.ShapeDtypeStruct((G, P), jnp.float32),
    )(ps, cnt, wp1, bp1, wp2, bp2)


def kernel(x, edge_index, batch, W1, b1, W2, b2, Wp1, bp1, Wp2, bp2):
    src = edge_index[0]
    dst = edge_index[1]
    batch3 = batch.reshape(NBLK, 1, BLK)
    agg1 = _sc_segment_sum(x, src, dst)
    h = _tc_layer1(x, agg1, W1, b1.reshape(1, D))
    agg2 = _sc_segment_sum(h, src, dst)
    ps, cnt = _tc_layer2(h, agg2, W2, b2.reshape(1, D), batch3)
    return _tc_head(ps, cnt, Wp1, bp1.reshape(1, D), Wp2, bp2.reshape(1, P))


# speedup vs baseline: 14.3406x; 1.4059x over previous
import functools

import jax
import jax.numpy as jnp
from jax import lax
from jax.experimental import pallas as pl
from jax.experimental.pallas import tpu as pltpu
from jax.experimental.pallas import tpu_sc as plsc

N = 10000
D = 128
G = 128
P = 64
E = 320000

NC = 2
NS = 16
NW = NC * NS
EPW = E // NW
CHUNK = 40
NCHUNK = EPW // CHUNK
NPAD = 10240
RPT = NPAD // NS
RING = 8

@functools.cache
def _sc_segment_sum_fn():
    mesh = plsc.VectorSubcoreMesh(
        core_axis_name="c", subcore_axis_name="s",
        num_cores=NC, num_subcores=NS)
    return functools.partial(
        pl.kernel,
        out_type=jax.ShapeDtypeStruct((NC, NPAD, D), jnp.float32),
        mesh=mesh,
        scratch_types=[
            pltpu.VMEM_SHARED((NPAD, D), jnp.float32),
            [pltpu.VMEM((CHUNK,), jnp.int32) for _ in range(RING)],
            [pltpu.VMEM((CHUNK,), jnp.int32) for _ in range(RING)],
            pltpu.VMEM((RING, CHUNK, D), jnp.float32),
            pltpu.SemaphoreType.DMA,
            pltpu.SemaphoreType.DMA,
            pltpu.SemaphoreType.DMA,
        ],
    )(_sc_segment_sum_body)


def _sc_segment_sum(feat, src, dst):
    return _sc_segment_sum_fn()(feat, src, dst)


def _sc_segment_sum_body(feat_hbm, src_hbm, dst_hbm, out_hbm,
                         acc_sh, sbufs, dbufs, rows_v, isem, gsem, ssem):
    c = lax.axis_index("c")
    s = lax.axis_index("s")
    wid = c * NS + s
    base = wid * EPW

    def _zb(i, carry):
        rows_v[0, i // 8, pl.ds((i % 8) * 16, 16)] = jnp.zeros((16,), jnp.float32)
        return carry
    lax.fori_loop(0, CHUNK * 8, _zb, 0)
    for t in range(RPT // CHUNK):
        pltpu.async_copy(rows_v.at[0],
                         acc_sh.at[pl.ds(s * RPT + t * CHUNK, CHUNK)], ssem)
    for t in range(RPT // CHUNK):
        pltpu.make_async_copy(rows_v.at[0],
                              acc_sh.at[pl.ds(s * RPT + t * CHUNK, CHUNK)],
                              ssem).wait()
    plsc.subcore_barrier()

    def _i_issue(j, r):
        off = base + j * CHUNK
        pltpu.async_copy(src_hbm.at[pl.ds(off, CHUNK)], sbufs[r], isem)
        pltpu.async_copy(dst_hbm.at[pl.ds(off, CHUNK)], dbufs[r], isem)

    def _i_drain(j, r):
        off = base + j * CHUNK
        pltpu.make_async_copy(src_hbm.at[pl.ds(off, CHUNK)], sbufs[r],
                              isem).wait()
        pltpu.make_async_copy(dst_hbm.at[pl.ds(off, CHUNK)], dbufs[r],
                              isem).wait()

    def _g_issue(r):
        pltpu.async_copy(feat_hbm.at[sbufs[r]], rows_v.at[r], gsem)

    def _g_drain(r):
        pltpu.make_async_copy(feat_hbm.at[sbufs[r]], rows_v.at[r],
                              gsem).wait()

    def _s_issue(r):
        pltpu.async_copy(rows_v.at[r], acc_sh.at[dbufs[r]], ssem, add=True)

    def _s_drain(r):
        pltpu.make_async_copy(rows_v.at[r], acc_sh.at[dbufs[r]],
                              ssem).wait()

    for j in range(6):
        _i_issue(j, j)
    for j in range(4):
        _i_drain(j, j)
        _g_issue(j)
    for j in range(2):
        _g_drain(j)
        _s_issue(j)
        _i_issue(j + 6, j + 6)
        _i_drain(j + 4, j + 4)
        _g_issue(j + 4)

    def _steady(k, carry):
        j0 = 2 + RING * k
        for u in range(RING):
            j = j0 + u
            _g_drain((2 + u) % RING)
            _s_issue((2 + u) % RING)
            _s_drain(u % RING)
            _i_issue(j + 6, u % RING)
            _i_drain(j + 4, (6 + u) % RING)
            _g_issue((6 + u) % RING)
        return carry
    lax.fori_loop(0, (NCHUNK - 2 - RING) // RING, _steady, 0)

    for j in range(NCHUNK - RING, NCHUNK):
        _g_drain(j % RING)
        _s_issue(j % RING)
        _s_drain((j - 2) % RING)
        if j + 6 < NCHUNK:
            _i_issue(j + 6, (j + 6) % RING)
        if j + 4 < NCHUNK:
            _i_drain(j + 4, (j + 4) % RING)
            _g_issue((j + 4) % RING)
    for j in range(NCHUNK - 2, NCHUNK):
        _s_drain(j % RING)

    plsc.subcore_barrier()
    pltpu.sync_copy(acc_sh.at[pl.ds(s * RPT, RPT)],
                    out_hbm.at[c, pl.ds(s * RPT, RPT)])


BLK = 1000
NBLK = N // BLK


def _tc_layer1_body(x_ref, a_ref, w_ref, b_ref, o_ref):
    acc = x_ref[...] + a_ref[0] + a_ref[1]
    h = jnp.dot(acc, w_ref[...], preferred_element_type=jnp.float32) + b_ref[...]
    o_ref[...] = jnp.maximum(h, 0.0)


def _tc_layer1(x, agg, w, b):
    return pl.pallas_call(
        _tc_layer1_body,
        grid=(NBLK,),
        in_specs=[
            pl.BlockSpec((BLK, D), lambda i: (i, 0)),
            pl.BlockSpec((NC, BLK, D), lambda i: (0, i, 0)),
            pl.BlockSpec((D, D), lambda i: (0, 0)),
            pl.BlockSpec((1, D), lambda i: (0, 0)),
        ],
        out_specs=pl.BlockSpec((BLK, D), lambda i: (i, 0)),
        out_shape=jax.ShapeDtypeStruct((N, D), jnp.float32),
    )(x, agg, w, b)


def _tc_layer2_body(h_ref, a_ref, w_ref, b_ref, batch_ref, ps_ref, cnt_ref):
    i = pl.program_id(0)
    acc = h_ref[...] + a_ref[0] + a_ref[1]
    h2 = jnp.maximum(
        jnp.dot(acc, w_ref[...], preferred_element_type=jnp.float32) + b_ref[...],
        0.0)
    b2d = batch_ref[0]
    onehot_t = (lax.broadcasted_iota(jnp.int32, (G, 1), 0) == b2d
                ).astype(jnp.float32)
    ps = jnp.dot(onehot_t, h2, preferred_element_type=jnp.float32)
    cnt = jnp.dot(onehot_t, jnp.ones((BLK, D), jnp.float32),
                  preferred_element_type=jnp.float32)

    @pl.when(i == 0)
    def _():
        ps_ref[...] = jnp.zeros_like(ps_ref)
        cnt_ref[...] = jnp.zeros_like(cnt_ref)

    ps_ref[...] += ps
    cnt_ref[...] += cnt


def _tc_layer2(h, agg, w, b, batch3):
    return pl.pallas_call(
        _tc_layer2_body,
        grid=(NBLK,),
        in_specs=[
            pl.BlockSpec((BLK, D), lambda i: (i, 0)),
            pl.BlockSpec((NC, BLK, D), lambda i: (0, i, 0)),
            pl.BlockSpec((D, D), lambda i: (0, 0)),
            pl.BlockSpec((1, D), lambda i: (0, 0)),
            pl.BlockSpec((1, 1, BLK), lambda i: (i, 0, 0)),
        ],
        out_specs=[
            pl.BlockSpec((G, D), lambda i: (0, 0)),
            pl.BlockSpec((G, D), lambda i: (0, 0)),
        ],
        out_shape=[
            jax.ShapeDtypeStruct((G, D), jnp.float32),
            jax.ShapeDtypeStruct((G, D), jnp.float32),
        ],
    )(h, agg, w, b, batch3)


def _tc_head_body(ps_ref, cnt_ref, wp1_ref, bp1_ref, wp2_ref, bp2_ref, z_ref):
    hg = ps_ref[...] / jnp.maximum(cnt_ref[...], 1.0)
    t = jnp.maximum(
        jnp.dot(hg, wp1_ref[...], preferred_element_type=jnp.float32)
        + bp1_ref[...], 0.0)
    z_ref[...] = (jnp.dot(t, wp2_ref[...], preferred_element_type=jnp.float32)
                  + bp2_ref[...])


def _tc_head(ps, cnt, wp1, bp1, wp2, bp2):
    return pl.pallas_call(
        _tc_head_body,
        out_shape=jax.ShapeDtypeStruct((G, P), jnp.float32),
    )(ps, cnt, wp1, bp1, wp2, bp2)


def kernel(x, edge_index, batch, W1, b1, W2, b2, Wp1, bp1, Wp2, bp2):
    src = edge_index[0]
    dst = edge_index[1]
    batch3 = batch.reshape(NBLK, 1, BLK)
    agg1 = _sc_segment_sum(x, src, dst)
    h = _tc_layer1(x, agg1, W1, b1.reshape(1, D))
    agg2 = _sc_segment_sum(h, src, dst)
    ps, cnt = _tc_layer2(h, agg2, W2, b2.reshape(1, D), batch3)
    return _tc_head(ps, cnt, Wp1, bp1.reshape(1, D), Wp2, bp2.reshape(1, P))
